# Initial kernel scaffold; baseline (speedup 1.0000x reference)
import jax
import jax.numpy as jnp
from jax.experimental import pallas as pl


def kernel(d, Angles, edge_index_1):
    raise NotImplementedError("write your pallas kernel here")


# speedup vs baseline: 1.1673x; 1.1673x over previous
import functools

import jax
import jax.numpy as jnp
import numpy as np
from jax import lax
from jax.experimental import pallas as pl
from jax.experimental.pallas import tpu as pltpu

NUM_SPHERICAL = 7
NUM_RADIAL = 6
CUTOFF = 5.0
EXPONENT = 5
NCOL = NUM_SPHERICAL * NUM_RADIAL
PACK = 2
LANES = NCOL * PACK


def _jn_np(x, n):
    x = np.asarray(x, dtype=np.float64)
    jm1 = np.sin(x) / x
    if n == 0:
        return jm1
    jc = np.sin(x) / x ** 2 - np.cos(x) / x
    for l in range(2, n + 1):
        jm1, jc = jc, (2 * l - 1) / x * jc - jm1
    return jc


def _jn_zeros(n, k):
    zerosj = np.zeros((n, k), dtype=np.float64)
    zerosj[0] = np.arange(1, k + 1) * np.pi
    points = np.arange(1, k + n) * np.pi
    racines = np.zeros(k + n - 1, dtype=np.float64)
    for i in range(1, n):
        for j in range(k + n - 1 - i):
            a, b = float(points[j]), float(points[j + 1])
            fa = _jn_np(a, i)
            for _ in range(100):
                m = 0.5 * (a + b)
                fm = _jn_np(m, i)
                if np.sign(fm) == np.sign(fa):
                    a, fa = m, fm
                else:
                    b = m
            racines[j] = 0.5 * (a + b)
        points = racines.copy()
        zerosj[i, :k] = racines[:k]
    return zerosj


_ZEROS = _jn_zeros(NUM_SPHERICAL, NUM_RADIAL)
_NORMS = np.zeros((NUM_SPHERICAL, NUM_RADIAL), dtype=np.float64)
for _l in range(NUM_SPHERICAL):
    for _k in range(NUM_RADIAL):
        _NORMS[_l, _k] = 1.0 / np.sqrt(0.5 * _jn_np(_ZEROS[_l, _k], _l + 1) ** 2)

_COEF_L = np.sqrt((2.0 * np.arange(NUM_SPHERICAL) + 1.0) / (4.0 * np.pi))
_Z_LANE = np.tile(_ZEROS.reshape(NCOL), PACK).astype(np.float32)
_N_LANE = np.tile(
    (_NORMS * _COEF_L[:, None]).reshape(NCOL), PACK
).astype(np.float32)
_L_LANE = np.tile(np.repeat(np.arange(NUM_SPHERICAL), NUM_RADIAL), PACK).astype(
    np.float32
)

_BLOCK = 640


def _fb2d_block(d2_ref, a2_ref, z_ref, n_ref, l_ref, out_ref):
    b = d2_ref.shape[0]
    lane = lax.broadcasted_iota(jnp.int32, (b, LANES), 1)
    first = lane < NCOL

    x0 = d2_ref[:, 0:1] * (1.0 / CUTOFF)
    x1 = d2_ref[:, 1:2] * (1.0 / CUTOFF)
    x = jnp.where(first, x0, x1)
    a0 = a2_ref[:, 0:1]
    a1 = a2_ref[:, 1:2]
    ang = jnp.where(first, a0, a1)

    z = z_ref[0:1, :]
    nrm = n_ref[0:1, :]
    ll = l_ref[0:1, :]

    t = x * z
    s = jnp.sin(t)
    c = jnp.cos(t)
    r = 1.0 / t
    sr = s * r
    js = [sr, (sr - c) * r]
    for l in range(2, NUM_SPHERICAL):
        js.append((2 * l - 1) * r * js[l - 1] - js[l - 2])
    jsel = js[NUM_SPHERICAL - 1]
    for l in range(NUM_SPHERICAL - 2, -1, -1):
        jsel = jnp.where(ll == float(l), js[l], jsel)

    p = EXPONENT + 1
    ea = -(p + 1) * (p + 2) / 2.0
    eb = p * (p + 2.0)
    ec = -p * (p + 1) / 2.0
    x5 = (x * x) * (x * x) * x
    env = (1.0 / x + x5 * (ea + x * (eb + x * ec))) * (x < 1.0).astype(jnp.float32)

    ct = jnp.cos(ang)
    ps = [jnp.ones_like(ct), ct]
    for l in range(2, NUM_SPHERICAL):
        ps.append(
            ((2 * l - 1) / l) * ct * ps[l - 1] - ((l - 1) / l) * ps[l - 2]
        )
    psel = ps[NUM_SPHERICAL - 1]
    for l in range(NUM_SPHERICAL - 2, -1, -1):
        psel = jnp.where(ll == float(l), ps[l], psel)

    out_ref[...] = (env * nrm) * jsel * psel


def _dense_eval(d_g, angles, interpret=False):
    t = d_g.shape[0]
    rows = t // PACK
    d2 = d_g.reshape(rows, PACK)
    a2 = angles.reshape(rows, PACK)
    z = jnp.asarray(_Z_LANE).reshape(1, LANES)
    n = jnp.asarray(_N_LANE).reshape(1, LANES)
    l = jnp.asarray(_L_LANE).reshape(1, LANES)
    grid = (rows + _BLOCK - 1) // _BLOCK
    out = pl.pallas_call(
        _fb2d_block,
        grid=(grid,),
        in_specs=[
            pl.BlockSpec((_BLOCK, PACK), lambda i: (i, 0)),
            pl.BlockSpec((_BLOCK, PACK), lambda i: (i, 0)),
            pl.BlockSpec((1, LANES), lambda i: (0, 0)),
            pl.BlockSpec((1, LANES), lambda i: (0, 0)),
            pl.BlockSpec((1, LANES), lambda i: (0, 0)),
        ],
        out_specs=pl.BlockSpec((_BLOCK, LANES), lambda i: (i, 0)),
        out_shape=jax.ShapeDtypeStruct((rows, LANES), jnp.float32),
        compiler_params=pltpu.CompilerParams(
            dimension_semantics=("parallel",)
        ),
        interpret=interpret,
    )(d2, a2, z, n, l)
    return out.reshape(t, NCOL)


@jax.jit
def kernel(d, Angles, edge_index_1):
    d_g = jnp.take(d, edge_index_1)
    return _dense_eval(d_g, Angles)


# speedup vs baseline: 1.9155x; 1.6410x over previous
import functools

import jax
import jax.numpy as jnp
import numpy as np
from jax import lax
from jax.experimental import pallas as pl
from jax.experimental.pallas import tpu as pltpu

NUM_SPHERICAL = 7
NUM_RADIAL = 6
CUTOFF = 5.0
EXPONENT = 5
NCOL = NUM_SPHERICAL * NUM_RADIAL
PACK = 2
LANES = NCOL * PACK


def _jn_np(x, n):
    x = np.asarray(x, dtype=np.float64)
    jm1 = np.sin(x) / x
    if n == 0:
        return jm1
    jc = np.sin(x) / x ** 2 - np.cos(x) / x
    for l in range(2, n + 1):
        jm1, jc = jc, (2 * l - 1) / x * jc - jm1
    return jc


def _jn_zeros(n, k):
    zerosj = np.zeros((n, k), dtype=np.float64)
    zerosj[0] = np.arange(1, k + 1) * np.pi
    points = np.arange(1, k + n) * np.pi
    racines = np.zeros(k + n - 1, dtype=np.float64)
    for i in range(1, n):
        for j in range(k + n - 1 - i):
            a, b = float(points[j]), float(points[j + 1])
            fa = _jn_np(a, i)
            for _ in range(100):
                m = 0.5 * (a + b)
                fm = _jn_np(m, i)
                if np.sign(fm) == np.sign(fa):
                    a, fa = m, fm
                else:
                    b = m
            racines[j] = 0.5 * (a + b)
        points = racines.copy()
        zerosj[i, :k] = racines[:k]
    return zerosj


_ZEROS = _jn_zeros(NUM_SPHERICAL, NUM_RADIAL)
_NORMS = np.zeros((NUM_SPHERICAL, NUM_RADIAL), dtype=np.float64)
for _l in range(NUM_SPHERICAL):
    for _k in range(NUM_RADIAL):
        _NORMS[_l, _k] = 1.0 / np.sqrt(0.5 * _jn_np(_ZEROS[_l, _k], _l + 1) ** 2)


def _bessel_uv_coeffs():
    f = [np.array([0.0, 1.0])]
    g = [np.array([0.0])]
    f.append(np.array([0.0, 0.0, 1.0]))
    g.append(np.array([0.0, -1.0]))

    def shift(p):
        return np.concatenate([[0.0], p])

    def sub(p, q):
        n = max(len(p), len(q))
        r = np.zeros(n)
        r[: len(p)] += p
        r[: len(q)] -= q
        return r

    for l in range(2, NUM_SPHERICAL):
        f.append(sub((2 * l - 1) * shift(f[l - 1]), f[l - 2]))
        g.append(sub((2 * l - 1) * shift(g[l - 1]), g[l - 2]))
    return f, g


def _legendre_coeffs():
    ps = [np.array([1.0]), np.array([0.0, 1.0])]
    for l in range(2, NUM_SPHERICAL):
        a = np.concatenate([[0.0], ps[l - 1]]) * (2 * l - 1) / l
        b = np.zeros(len(a))
        b[: len(ps[l - 2])] = ps[l - 2] * (l - 1) / l
        ps.append(a - b)
    return ps


_COEF_L = np.sqrt((2.0 * np.arange(NUM_SPHERICAL) + 1.0) / (4.0 * np.pi))

_DEG_P = NUM_SPHERICAL + 1
_DEG_Q = NUM_SPHERICAL
_DEG_L = NUM_SPHERICAL


def _lane_constants():
    fs, gs = _bessel_uv_coeffs()
    legs = _legendre_coeffs()
    pb = np.zeros((_DEG_P, LANES))
    qb = np.zeros((_DEG_Q, LANES))
    lg = np.zeros((_DEG_L, LANES))
    zl = np.zeros(LANES)
    izl = np.zeros(LANES)
    for j in range(LANES):
        c = j % NCOL
        l, k = c // NUM_RADIAL, c % NUM_RADIAL
        nrm = _NORMS[l, k]
        for i, v in enumerate(fs[l]):
            pb[i, j] = nrm * v
        for i, v in enumerate(gs[l]):
            qb[i, j] = nrm * v
        for i, v in enumerate(legs[l]):
            lg[i, j] = _COEF_L[l] * v
        zl[j] = _ZEROS[l, k]
        izl[j] = 1.0 / _ZEROS[l, k]
    rows = [pb[i] for i in range(_DEG_P - 1, -1, -1)]
    rows += [qb[i] for i in range(_DEG_Q - 1, -1, -1)]
    rows += [lg[i] for i in range(_DEG_L - 1, -1, -1)]
    rows += [zl, izl]
    pad = 32 - len(rows)
    rows += [np.zeros(LANES)] * pad
    return np.stack(rows).astype(np.float32)


_CONSTS = _lane_constants()
_ROW_P = 0
_ROW_Q = _DEG_P
_ROW_L = _DEG_P + _DEG_Q
_ROW_Z = _ROW_L + _DEG_L
_ROW_IZ = _ROW_Z + 1

_BLOCK = 640

_INV_PIO2 = 0.6366197723675814
_RND_MAGIC = 12582912.0
_D1 = 1.5703125
_D2 = 4.837512969970703125e-4
_D3 = 7.54978995489188154e-8
_SIN_C = (-1.9515295891e-4, 8.3321608736e-3, -1.6666654611e-1)
_COS_C = (2.443315711809948e-5, -1.388731625493765e-3, 4.166664568298827e-2)
_SINW_C = (-2.5052108e-8, 2.7557319e-6, -1.98412698e-4, 8.3333333e-3,
           -0.16666667)
_PIO2 = 1.5707963267948966


def _fb2d_block(d2_ref, a2_ref, c_ref, out_ref):
    b = d2_ref.shape[0]
    lane = lax.broadcasted_iota(jnp.int32, (b, LANES), 1)
    first = lane < NCOL

    x0 = d2_ref[:, 0:1] * (1.0 / CUTOFF)
    x1 = d2_ref[:, 1:2] * (1.0 / CUTOFF)
    x = jnp.where(first, x0, x1)
    a0 = a2_ref[:, 0:1]
    a1 = a2_ref[:, 1:2]
    ang = jnp.where(first, a0, a1)

    z = c_ref[_ROW_Z : _ROW_Z + 1, :]
    iz = c_ref[_ROW_IZ : _ROW_IZ + 1, :]

    t = x * z
    ki = (t * _INV_PIO2 + 0.5).astype(jnp.int32)
    kf = ki.astype(jnp.float32)
    r = ((t - kf * _D1) - kf * _D2) - kf * _D3
    zz = r * r
    sp = ((_SIN_C[0] * zz + _SIN_C[1]) * zz + _SIN_C[2]) * zz * r + r
    cp = ((_COS_C[0] * zz + _COS_C[1]) * zz + _COS_C[2]) * zz * zz - 0.5 * zz + 1.0
    swap = (ki & 1) == 1
    ssel = jnp.where(swap, cp, sp)
    csel = jnp.where(swap, sp, cp)
    ssign = (ki & 2) << 30
    csign = ((ki + 1) & 2) << 30
    s = lax.bitcast_convert_type(
        lax.bitcast_convert_type(ssel, jnp.int32) ^ ssign, jnp.float32
    )
    c = lax.bitcast_convert_type(
        lax.bitcast_convert_type(csel, jnp.int32) ^ csign, jnp.float32
    )

    invx = 1.0 / x
    u = invx * iz
    p = c_ref[_ROW_P : _ROW_P + 1, :]
    for i in range(1, _DEG_P):
        p = p * u + c_ref[_ROW_P + i : _ROW_P + i + 1, :]
    q = c_ref[_ROW_Q : _ROW_Q + 1, :]
    for i in range(1, _DEG_Q):
        q = q * u + c_ref[_ROW_Q + i : _ROW_Q + i + 1, :]
    radial = s * p + c * q

    pp = EXPONENT + 1
    ea = -(pp + 1) * (pp + 2) / 2.0
    eb = pp * (pp + 2.0)
    ec = -pp * (pp + 1) / 2.0
    x2 = x * x
    x5 = x2 * x2 * x
    env = invx + x5 * (ea + x * (eb + x * ec))
    env = jnp.where(x < 1.0, env, 0.0)

    w = _PIO2 - ang
    wz = w * w
    ct = (
        ((((_SINW_C[0] * wz + _SINW_C[1]) * wz + _SINW_C[2]) * wz + _SINW_C[3])
         * wz + _SINW_C[4]) * wz * w + w
    )
    lg = c_ref[_ROW_L : _ROW_L + 1, :]
    for i in range(1, _DEG_L):
        lg = lg * ct + c_ref[_ROW_L + i : _ROW_L + i + 1, :]

    out_ref[...] = (env * radial) * lg


def _dense_eval(d_g, angles, interpret=False):
    t = d_g.shape[0]
    rows = t // PACK
    d2 = d_g.reshape(rows, PACK)
    a2 = angles.reshape(rows, PACK)
    consts = jnp.asarray(_CONSTS)
    grid = (rows + _BLOCK - 1) // _BLOCK
    out = pl.pallas_call(
        _fb2d_block,
        grid=(grid,),
        in_specs=[
            pl.BlockSpec((_BLOCK, PACK), lambda i: (i, 0)),
            pl.BlockSpec((_BLOCK, PACK), lambda i: (i, 0)),
            pl.BlockSpec((32, LANES), lambda i: (0, 0)),
        ],
        out_specs=pl.BlockSpec((_BLOCK, LANES), lambda i: (i, 0)),
        out_shape=jax.ShapeDtypeStruct((rows, LANES), jnp.float32),
        compiler_params=pltpu.CompilerParams(
            dimension_semantics=("parallel",)
        ),
        interpret=interpret,
    )(d2, a2, consts)
    return out.reshape(t, NCOL)


@jax.jit
def kernel(d, Angles, edge_index_1):
    d_g = jnp.take(d, edge_index_1)
    return _dense_eval(d_g, Angles)


# speedup vs baseline: 1.9244x; 1.0046x over previous
import functools

import jax
import jax.numpy as jnp
import numpy as np
from jax import lax
from jax.experimental import pallas as pl
from jax.experimental.pallas import tpu as pltpu
from jax.experimental.pallas import tpu_sc as plsc

NUM_SPHERICAL = 7
NUM_RADIAL = 6
CUTOFF = 5.0
EXPONENT = 5
NCOL = NUM_SPHERICAL * NUM_RADIAL
PACK = 2
LANES = NCOL * PACK


def _jn_np(x, n):
    x = np.asarray(x, dtype=np.float64)
    jm1 = np.sin(x) / x
    if n == 0:
        return jm1
    jc = np.sin(x) / x ** 2 - np.cos(x) / x
    for l in range(2, n + 1):
        jm1, jc = jc, (2 * l - 1) / x * jc - jm1
    return jc


def _jn_zeros(n, k):
    zerosj = np.zeros((n, k), dtype=np.float64)
    zerosj[0] = np.arange(1, k + 1) * np.pi
    points = np.arange(1, k + n) * np.pi
    racines = np.zeros(k + n - 1, dtype=np.float64)
    for i in range(1, n):
        for j in range(k + n - 1 - i):
            a, b = float(points[j]), float(points[j + 1])
            fa = _jn_np(a, i)
            for _ in range(100):
                m = 0.5 * (a + b)
                fm = _jn_np(m, i)
                if np.sign(fm) == np.sign(fa):
                    a, fa = m, fm
                else:
                    b = m
            racines[j] = 0.5 * (a + b)
        points = racines.copy()
        zerosj[i, :k] = racines[:k]
    return zerosj


_ZEROS = _jn_zeros(NUM_SPHERICAL, NUM_RADIAL)
_NORMS = np.zeros((NUM_SPHERICAL, NUM_RADIAL), dtype=np.float64)
for _l in range(NUM_SPHERICAL):
    for _k in range(NUM_RADIAL):
        _NORMS[_l, _k] = 1.0 / np.sqrt(0.5 * _jn_np(_ZEROS[_l, _k], _l + 1) ** 2)


def _bessel_uv_coeffs():
    f = [np.array([0.0, 1.0])]
    g = [np.array([0.0])]
    f.append(np.array([0.0, 0.0, 1.0]))
    g.append(np.array([0.0, -1.0]))

    def shift(p):
        return np.concatenate([[0.0], p])

    def sub(p, q):
        n = max(len(p), len(q))
        r = np.zeros(n)
        r[: len(p)] += p
        r[: len(q)] -= q
        return r

    for l in range(2, NUM_SPHERICAL):
        f.append(sub((2 * l - 1) * shift(f[l - 1]), f[l - 2]))
        g.append(sub((2 * l - 1) * shift(g[l - 1]), g[l - 2]))
    return f, g


def _legendre_coeffs():
    ps = [np.array([1.0]), np.array([0.0, 1.0])]
    for l in range(2, NUM_SPHERICAL):
        a = np.concatenate([[0.0], ps[l - 1]]) * (2 * l - 1) / l
        b = np.zeros(len(a))
        b[: len(ps[l - 2])] = ps[l - 2] * (l - 1) / l
        ps.append(a - b)
    return ps


_COEF_L = np.sqrt((2.0 * np.arange(NUM_SPHERICAL) + 1.0) / (4.0 * np.pi))

_DEG_P = NUM_SPHERICAL + 1
_DEG_Q = NUM_SPHERICAL
_DEG_L = NUM_SPHERICAL


def _lane_constants():
    fs, gs = _bessel_uv_coeffs()
    legs = _legendre_coeffs()
    pb = np.zeros((_DEG_P, LANES))
    qb = np.zeros((_DEG_Q, LANES))
    lg = np.zeros((_DEG_L, LANES))
    zl = np.zeros(LANES)
    izl = np.zeros(LANES)
    for j in range(LANES):
        c = j % NCOL
        l, k = c // NUM_RADIAL, c % NUM_RADIAL
        nrm = _NORMS[l, k]
        for i, v in enumerate(fs[l]):
            pb[i, j] = nrm * v
        for i, v in enumerate(gs[l]):
            qb[i, j] = nrm * v
        for i, v in enumerate(legs[l]):
            lg[i, j] = _COEF_L[l] * v
        zl[j] = _ZEROS[l, k]
        izl[j] = 1.0 / _ZEROS[l, k]
    rows = [pb[i] for i in range(_DEG_P - 1, -1, -1)]
    rows += [qb[i] for i in range(_DEG_Q - 1, -1, -1)]
    rows += [lg[i] for i in range(_DEG_L - 1, -1, -1)]
    rows += [zl, izl]
    pad = 32 - len(rows)
    rows += [np.zeros(LANES)] * pad
    return np.stack(rows).astype(np.float32)


_CONSTS = _lane_constants()
_ROW_P = 0
_ROW_Q = _DEG_P
_ROW_L = _DEG_P + _DEG_Q
_ROW_Z = _ROW_L + _DEG_L
_ROW_IZ = _ROW_Z + 1

_BLOCK = 640

_INV_PIO2 = 0.6366197723675814
_RND_MAGIC = 12582912.0
_D1 = 1.5703125
_D2 = 4.837512969970703125e-4
_D3 = 7.54978995489188154e-8
_SIN_C = (-1.9515295891e-4, 8.3321608736e-3, -1.6666654611e-1)
_COS_C = (2.443315711809948e-5, -1.388731625493765e-3, 4.166664568298827e-2)
_SINW_C = (-2.5052108e-8, 2.7557319e-6, -1.98412698e-4, 8.3333333e-3,
           -0.16666667)
_PIO2 = 1.5707963267948966


def _fb2d_block(d2_ref, a2_ref, c_ref, out_ref):
    b = d2_ref.shape[0]
    lane = lax.broadcasted_iota(jnp.int32, (b, LANES), 1)
    first = lane < NCOL

    x0 = d2_ref[:, 0:1] * (1.0 / CUTOFF)
    x1 = d2_ref[:, 1:2] * (1.0 / CUTOFF)
    x = jnp.where(first, x0, x1)
    a0 = a2_ref[:, 0:1]
    a1 = a2_ref[:, 1:2]
    ang = jnp.where(first, a0, a1)

    z = c_ref[_ROW_Z : _ROW_Z + 1, :]
    iz = c_ref[_ROW_IZ : _ROW_IZ + 1, :]

    t = x * z
    ki = (t * _INV_PIO2 + 0.5).astype(jnp.int32)
    kf = ki.astype(jnp.float32)
    r = ((t - kf * _D1) - kf * _D2) - kf * _D3
    zz = r * r
    sp = ((_SIN_C[0] * zz + _SIN_C[1]) * zz + _SIN_C[2]) * zz * r + r
    cp = ((_COS_C[0] * zz + _COS_C[1]) * zz + _COS_C[2]) * zz * zz - 0.5 * zz + 1.0
    swap = (ki & 1) == 1
    ssel = jnp.where(swap, cp, sp)
    csel = jnp.where(swap, sp, cp)
    ssign = (ki & 2) << 30
    csign = ((ki + 1) & 2) << 30
    s = lax.bitcast_convert_type(
        lax.bitcast_convert_type(ssel, jnp.int32) ^ ssign, jnp.float32
    )
    c = lax.bitcast_convert_type(
        lax.bitcast_convert_type(csel, jnp.int32) ^ csign, jnp.float32
    )

    invx = 1.0 / x
    u = invx * iz
    p = c_ref[_ROW_P : _ROW_P + 1, :]
    for i in range(1, _DEG_P):
        p = p * u + c_ref[_ROW_P + i : _ROW_P + i + 1, :]
    q = c_ref[_ROW_Q : _ROW_Q + 1, :]
    for i in range(1, _DEG_Q):
        q = q * u + c_ref[_ROW_Q + i : _ROW_Q + i + 1, :]
    radial = s * p + c * q

    pp = EXPONENT + 1
    ea = -(pp + 1) * (pp + 2) / 2.0
    eb = pp * (pp + 2.0)
    ec = -pp * (pp + 1) / 2.0
    x2 = x * x
    x5 = x2 * x2 * x
    env = invx + x5 * (ea + x * (eb + x * ec))
    env = jnp.where(x < 1.0, env, 0.0)

    w = _PIO2 - ang
    wz = w * w
    ct = (
        ((((_SINW_C[0] * wz + _SINW_C[1]) * wz + _SINW_C[2]) * wz + _SINW_C[3])
         * wz + _SINW_C[4]) * wz * w + w
    )
    lg = c_ref[_ROW_L : _ROW_L + 1, :]
    for i in range(1, _DEG_L):
        lg = lg * ct + c_ref[_ROW_L + i : _ROW_L + i + 1, :]

    out_ref[...] = (env * radial) * lg


def _dense_eval(d_g, angles, interpret=False):
    t = d_g.shape[0]
    rows = t // PACK
    d2 = d_g.reshape(rows, PACK)
    a2 = angles.reshape(rows, PACK)
    consts = jnp.asarray(_CONSTS)
    grid = (rows + _BLOCK - 1) // _BLOCK
    out = pl.pallas_call(
        _fb2d_block,
        grid=(grid,),
        in_specs=[
            pl.BlockSpec((_BLOCK, PACK), lambda i: (i, 0)),
            pl.BlockSpec((_BLOCK, PACK), lambda i: (i, 0)),
            pl.BlockSpec((32, LANES), lambda i: (0, 0)),
        ],
        out_specs=pl.BlockSpec((_BLOCK, LANES), lambda i: (i, 0)),
        out_shape=jax.ShapeDtypeStruct((rows, LANES), jnp.float32),
        compiler_params=pltpu.CompilerParams(
            dimension_semantics=("parallel",)
        ),
        interpret=interpret,
    )(d2, a2, consts)
    return out.reshape(t, NCOL)


def _sc_gather(d, idx):
    info = plsc.get_sparse_core_info()
    nw = info.num_cores * info.num_subcores
    per = idx.shape[0] // nw

    @functools.partial(
        pl.kernel,
        mesh=plsc.VectorSubcoreMesh(core_axis_name="c", subcore_axis_name="s"),
        out_type=jax.ShapeDtypeStruct(idx.shape, jnp.float32),
        scratch_types=[
            pltpu.VMEM((per,), jnp.int32),
            pltpu.VMEM((per,), jnp.float32),
            pltpu.SemaphoreType.DMA,
        ],
    )
    def gk(d_hbm, idx_hbm, out_hbm, idx_v, rows_v, sem):
        wid = lax.axis_index("s") * info.num_cores + lax.axis_index("c")
        base = wid * per
        pltpu.sync_copy(idx_hbm.at[pl.ds(base, per)], idx_v)
        pltpu.async_copy(d_hbm.at[idx_v], rows_v, sem).wait()
        pltpu.sync_copy(rows_v, out_hbm.at[pl.ds(base, per)])

    return gk(d, idx)


@jax.jit
def kernel(d, Angles, edge_index_1):
    d_g = _sc_gather(d, edge_index_1)
    return _dense_eval(d_g, Angles)
